# Initial kernel scaffold; baseline (speedup 1.0000x reference)
import jax
import jax.numpy as jnp
from jax.experimental import pallas as pl


def kernel(x, emb, A, bA, W, U, b, Wout, bout):
    raise NotImplementedError("write your pallas kernel here")


# speedup vs baseline: 11.5779x; 11.5779x over previous
import functools

import jax
import jax.numpy as jnp
from jax import lax
from jax.experimental import pallas as pl
from jax.experimental.pallas import tpu as pltpu
from jax.experimental.pallas import tpu_sc as plsc

_NUMSTEPS = 10


def _chunk_plan(b_per_w):
    for c in range(1, b_per_w + 1):
        if b_per_w % c == 0:
            k = b_per_w // c
            if k <= 128 and k % 8 == 0:
                return c, k
    raise ValueError(f"cannot chunk {b_per_w} rows per worker")


def _sc_gather(table, idx):
    v, d = table.shape
    (n,) = idx.shape
    info = plsc.get_sparse_core_info()
    nc, ns = info.num_cores, info.num_subcores
    nw = nc * ns
    assert n % nw == 0
    b_per_w = n // nw
    c_chunks, k_chunk = _chunk_plan(b_per_w)
    idx3 = idx.reshape(nw, c_chunks, k_chunk)
    mesh = plsc.VectorSubcoreMesh(core_axis_name="c", subcore_axis_name="s")

    @functools.partial(
        pl.kernel,
        mesh=mesh,
        out_type=jax.ShapeDtypeStruct((n, d), jnp.float32),
        scratch_types=[
            pltpu.VMEM((c_chunks, k_chunk), jnp.int32),
            pltpu.VMEM((b_per_w, d), jnp.float32),
            pltpu.SemaphoreType.DMA,
        ],
    )
    def gather_kernel(table_hbm, idx_hbm, out_hbm, idx_v, rows_v, sem):
        wid = lax.axis_index("s") * nc + lax.axis_index("c")
        pltpu.sync_copy(idx_hbm.at[wid], idx_v)
        copies = [
            pltpu.async_copy(
                table_hbm.at[idx_v.at[j]],
                rows_v.at[pl.ds(j * k_chunk, k_chunk)],
                sem,
            )
            for j in range(c_chunks)
        ]
        for cp in copies:
            cp.wait()
        pltpu.sync_copy(rows_v, out_hbm.at[pl.ds(wid * b_per_w, b_per_w)])

    return gather_kernel(table, idx3)


def _steps_body(wn, nsteps, h_ref, ac_ref, bac_ref, wc_ref, uc_ref, bc_ref,
                out_ref):
    h = h_ref[...]
    r_rows, hd = h.shape
    ac = ac_ref[...]
    bac = bac_ref[...]
    wc = wc_ref[...]
    uc = uc_ref[...]
    bc = bc_ref[...]
    row = lax.broadcasted_iota(jnp.int32, (r_rows, 1), 0)
    jpos = row % wn
    not_first = (jpos != 0).astype(h.dtype)
    not_last = (jpos != wn - 1).astype(h.dtype)
    zrow = jnp.zeros((1, 3 * hd), h.dtype)

    for _ in range(nsteps):
        tr = jnp.dot(h, ac, preferred_element_type=jnp.float32) + bac
        fwd = jnp.concatenate([zrow[:, :hd], tr[:-1, :hd]], axis=0) * not_first
        bwd = jnp.concatenate([tr[1:, hd : 2 * hd], zrow[:, :hd]], axis=0) * not_last
        agg = tr[:, 2 * hd :] + fwd + bwd
        gw = jnp.dot(agg, wc, preferred_element_type=jnp.float32) + bc
        gu = jnp.dot(h, uc, preferred_element_type=jnp.float32)
        r = jax.nn.sigmoid(gw[:, :hd] + gu[:, :hd])
        z = jax.nn.sigmoid(gw[:, hd : 2 * hd] + gu[:, hd : 2 * hd])
        nn = jnp.tanh(gw[:, 2 * hd :] + r * gu[:, 2 * hd :])
        h = (1.0 - z) * nn + z * h
    out_ref[...] = h


def _tc_steps(h0, ac, bac, wc, uc, bc, wn, nsteps):
    n, hd = h0.shape
    return pl.pallas_call(
        functools.partial(_steps_body, wn, nsteps),
        out_shape=jax.ShapeDtypeStruct((n, hd), jnp.float32),
    )(h0, ac, bac, wc, uc, bc)


def _proj_body(l_ref, w_ref, b_ref, out_ref):
    out_ref[...] = (
        jnp.dot(l_ref[...], w_ref[...], preferred_element_type=jnp.float32)
        + b_ref[...]
    )


def _tc_proj(last, wout, bout):
    bsz, hd = last.shape
    _, vocab = wout.shape
    vb = 2048
    grid = (vocab + vb - 1) // vb
    return pl.pallas_call(
        _proj_body,
        grid=(grid,),
        in_specs=[
            pl.BlockSpec((bsz, hd), lambda i: (0, 0)),
            pl.BlockSpec((hd, vb), lambda i: (0, i)),
            pl.BlockSpec((1, vb), lambda i: (0, i)),
        ],
        out_specs=pl.BlockSpec((bsz, vb), lambda i: (0, i)),
        out_shape=jax.ShapeDtypeStruct((bsz, vocab), jnp.float32),
    )(last, wout, bout.reshape(1, vocab))


def kernel(x, emb, A, bA, W, U, b, Wout, bout):
    bsz, seqlen = x.shape
    _, hd = emb.shape
    wn = min(seqlen, _NUMSTEPS + 1)
    xw = x[:, seqlen - wn :].reshape(-1).astype(jnp.int32)
    h0 = _sc_gather(emb, xw)
    ac = jnp.concatenate([A[1], A[2], A[3]], axis=1)
    bac = jnp.concatenate([bA[1], bA[2], bA[3]], axis=0).reshape(1, 3 * hd)
    wc = jnp.concatenate([W[0], W[1], W[2]], axis=1)
    uc = jnp.concatenate([U[0], U[1], U[2]], axis=1)
    bc = jnp.concatenate([b[0], b[1], b[2]], axis=0).reshape(1, 3 * hd)
    hfin = _tc_steps(h0, ac, bac, wc, uc, bc, wn, _NUMSTEPS)
    last = hfin.reshape(bsz, wn, hd)[:, -1, :]
    return _tc_proj(last, Wout, bout)


# speedup vs baseline: 12.6279x; 1.0907x over previous
import functools

import jax
import jax.numpy as jnp
from jax import lax
from jax.experimental import pallas as pl
from jax.experimental.pallas import tpu as pltpu
from jax.experimental.pallas import tpu_sc as plsc

_NUMSTEPS = 10


def _chunk_plan(b_per_w):
    for c in range(1, b_per_w + 1):
        if b_per_w % c == 0:
            k = b_per_w // c
            if k <= 128 and k % 8 == 0:
                return c, k
    raise ValueError(f"cannot chunk {b_per_w} rows per worker")


def _sc_gather(table, idx):
    v, d = table.shape
    (n,) = idx.shape
    info = plsc.get_sparse_core_info()
    nc, ns = info.num_cores, info.num_subcores
    nw = nc * ns
    assert n % nw == 0
    b_per_w = n // nw
    c_chunks, k_chunk = _chunk_plan(b_per_w)
    idx3 = idx.reshape(nw, c_chunks, k_chunk)
    mesh = plsc.VectorSubcoreMesh(core_axis_name="c", subcore_axis_name="s")

    @functools.partial(
        pl.kernel,
        mesh=mesh,
        out_type=jax.ShapeDtypeStruct((n, d), jnp.float32),
        scratch_types=[
            pltpu.VMEM((c_chunks, k_chunk), jnp.int32),
            pltpu.VMEM((b_per_w, d), jnp.float32),
            pltpu.SemaphoreType.DMA,
        ],
    )
    def gather_kernel(table_hbm, idx_hbm, out_hbm, idx_v, rows_v, sem):
        wid = lax.axis_index("s") * nc + lax.axis_index("c")
        pltpu.sync_copy(idx_hbm.at[wid], idx_v)
        copies = [
            pltpu.async_copy(
                table_hbm.at[idx_v.at[j]],
                rows_v.at[pl.ds(j * k_chunk, k_chunk)],
                sem,
            )
            for j in range(c_chunks)
        ]
        for cp in copies:
            cp.wait()
        pltpu.sync_copy(rows_v, out_hbm.at[pl.ds(wid * b_per_w, b_per_w)])

    return gather_kernel(table, idx3)


def _steps_body(bsz, nsteps, h_ref, ac_ref, bac_ref, wc_ref, uc_ref, bc_ref,
                out_ref):
    h_act = h_ref[...]
    hd = h_act.shape[1]
    ac = ac_ref[...]
    bac = bac_ref[...]
    wc = wc_ref[...]
    uc = uc_ref[...]
    bc = bc_ref[...]
    zblk = jnp.zeros((bsz, hd), h_act.dtype)

    for _ in range(nsteps):
        tr = jnp.dot(h_act, ac, preferred_element_type=jnp.float32) + bac
        hs = h_act[bsz:]
        m = hs.shape[0]
        fwd = tr[:m, :hd]
        if m > bsz:
            bwd = jnp.concatenate([tr[2 * bsz :, hd : 2 * hd], zblk], axis=0)
        else:
            bwd = zblk
        agg = tr[bsz:, 2 * hd :] + fwd + bwd
        gw = jnp.dot(agg, wc, preferred_element_type=jnp.float32) + bc
        gu = jnp.dot(hs, uc, preferred_element_type=jnp.float32)
        r = jax.nn.sigmoid(gw[:, :hd] + gu[:, :hd])
        z = jax.nn.sigmoid(gw[:, hd : 2 * hd] + gu[:, hd : 2 * hd])
        nn = jnp.tanh(gw[:, 2 * hd :] + r * gu[:, 2 * hd :])
        h_act = (1.0 - z) * nn + z * hs
    out_ref[...] = h_act


def _tc_steps(h0, ac, bac, wc, uc, bc, bsz, nsteps):
    n, hd = h0.shape
    assert n == bsz * (nsteps + 1)
    return pl.pallas_call(
        functools.partial(_steps_body, bsz, nsteps),
        out_shape=jax.ShapeDtypeStruct((bsz, hd), jnp.float32),
    )(h0, ac, bac, wc, uc, bc)


def _proj_body(l_ref, w_ref, b_ref, out_ref):
    out_ref[...] = (
        jnp.dot(l_ref[...], w_ref[...], preferred_element_type=jnp.float32)
        + b_ref[...]
    )


def _tc_proj(last, wout, bout):
    bsz, hd = last.shape
    _, vocab = wout.shape
    vb = 2048
    grid = (vocab + vb - 1) // vb
    return pl.pallas_call(
        _proj_body,
        grid=(grid,),
        in_specs=[
            pl.BlockSpec((bsz, hd), lambda i: (0, 0)),
            pl.BlockSpec((hd, vb), lambda i: (0, i)),
            pl.BlockSpec((1, vb), lambda i: (0, i)),
        ],
        out_specs=pl.BlockSpec((bsz, vb), lambda i: (0, i)),
        out_shape=jax.ShapeDtypeStruct((bsz, vocab), jnp.float32),
    )(last, wout, bout.reshape(1, vocab))


def kernel(x, emb, A, bA, W, U, b, Wout, bout):
    bsz, seqlen = x.shape
    _, hd = emb.shape
    wn = _NUMSTEPS + 1
    assert seqlen >= wn
    xw = x[:, seqlen - wn :].T.reshape(-1).astype(jnp.int32)
    h0 = _sc_gather(emb, xw)
    ac = jnp.concatenate([A[1], A[2], A[3]], axis=1)
    bac = jnp.concatenate([bA[1], bA[2], bA[3]], axis=0).reshape(1, 3 * hd)
    wc = jnp.concatenate([W[0], W[1], W[2]], axis=1)
    uc = jnp.concatenate([U[0], U[1], U[2]], axis=1)
    bc = jnp.concatenate([b[0], b[1], b[2]], axis=0).reshape(1, 3 * hd)
    last = _tc_steps(h0, ac, bac, wc, uc, bc, bsz, _NUMSTEPS)
    return _tc_proj(last, Wout, bout)


# speedup vs baseline: 12.9997x; 1.0294x over previous
import functools

import jax
import jax.numpy as jnp
from jax import lax
from jax.experimental import pallas as pl
from jax.experimental.pallas import tpu as pltpu
from jax.experimental.pallas import tpu_sc as plsc

_NUMSTEPS = 10


def _chunk_plan(b_per_w):
    for c in range(1, b_per_w + 1):
        if b_per_w % c == 0:
            k = b_per_w // c
            if k <= 128 and k % 8 == 0:
                return c, k
    raise ValueError(f"cannot chunk {b_per_w} rows per worker")


def _sc_gather(table, idx):
    v, d = table.shape
    (n,) = idx.shape
    info = plsc.get_sparse_core_info()
    nc, ns = info.num_cores, info.num_subcores
    nw = nc * ns
    assert n % nw == 0
    b_per_w = n // nw
    c_chunks, k_chunk = _chunk_plan(b_per_w)
    idx3 = idx.reshape(nw, c_chunks, k_chunk)
    mesh = plsc.VectorSubcoreMesh(core_axis_name="c", subcore_axis_name="s")

    @functools.partial(
        pl.kernel,
        mesh=mesh,
        out_type=jax.ShapeDtypeStruct((n, d), jnp.float32),
        scratch_types=[
            pltpu.VMEM((c_chunks, k_chunk), jnp.int32),
            pltpu.VMEM((b_per_w, d), jnp.float32),
            pltpu.SemaphoreType.DMA,
        ],
    )
    def gather_kernel(table_hbm, idx_hbm, out_hbm, idx_v, rows_v, sem):
        wid = lax.axis_index("s") * nc + lax.axis_index("c")
        pltpu.sync_copy(idx_hbm.at[wid], idx_v)
        copies = [
            pltpu.async_copy(
                table_hbm.at[idx_v.at[j]],
                rows_v.at[pl.ds(j * k_chunk, k_chunk)],
                sem,
            )
            for j in range(c_chunks)
        ]
        for cp in copies:
            cp.wait()
        pltpu.sync_copy(rows_v, out_hbm.at[pl.ds(wid * b_per_w, b_per_w)])

    return gather_kernel(table, idx3)


def _steps_body(bsz, nsteps, h_ref, ac_ref, bac_ref, wc_ref, uc_ref, bc_ref,
                out_ref):
    h_act = h_ref[...]
    hd = h_act.shape[1]
    ac = ac_ref[...]
    bac = bac_ref[...]
    wc = wc_ref[...]
    uc = uc_ref[...]
    bc = bc_ref[...]
    zblk = jnp.zeros((bsz, hd), h_act.dtype)

    for _ in range(nsteps):
        tr = jnp.dot(h_act, ac, preferred_element_type=jnp.float32) + bac
        hs = h_act[bsz:]
        m = hs.shape[0]
        fwd = tr[:m, :hd]
        if m > bsz:
            bwd = jnp.concatenate([tr[2 * bsz :, hd : 2 * hd], zblk], axis=0)
        else:
            bwd = zblk
        agg = tr[bsz:, 2 * hd :] + fwd + bwd
        gw = jnp.dot(agg, wc, preferred_element_type=jnp.float32) + bc
        gu = jnp.dot(hs, uc, preferred_element_type=jnp.float32)
        r = jax.nn.sigmoid(gw[:, :hd] + gu[:, :hd])
        z = jax.nn.sigmoid(gw[:, hd : 2 * hd] + gu[:, hd : 2 * hd])
        nn = jnp.tanh(gw[:, 2 * hd :] + r * gu[:, 2 * hd :])
        h_act = (1.0 - z) * nn + z * hs
    out_ref[...] = h_act


def _tc_steps(h0, ac, bac, wc, uc, bc, bsz, nsteps):
    n, hd = h0.shape
    assert n == bsz * (nsteps + 1)
    return pl.pallas_call(
        functools.partial(_steps_body, bsz, nsteps),
        out_shape=jax.ShapeDtypeStruct((bsz, hd), jnp.float32),
    )(h0, ac, bac, wc, uc, bc)


def _proj_body(l_ref, w_ref, b_ref, out_ref):
    out_ref[...] = (
        jnp.dot(l_ref[...], w_ref[...], preferred_element_type=jnp.float32)
        + b_ref[...]
    )


def _tc_proj(last, wout, bout):
    bsz, hd = last.shape
    _, vocab = wout.shape
    vb = 8192
    grid = (vocab + vb - 1) // vb
    return pl.pallas_call(
        _proj_body,
        grid=(grid,),
        in_specs=[
            pl.BlockSpec((bsz, hd), lambda i: (0, 0)),
            pl.BlockSpec((hd, vb), lambda i: (0, i)),
            pl.BlockSpec((1, vb), lambda i: (0, i)),
        ],
        out_specs=pl.BlockSpec((bsz, vb), lambda i: (0, i)),
        out_shape=jax.ShapeDtypeStruct((bsz, vocab), jnp.float32),
    )(last, wout, bout.reshape(1, vocab))


def kernel(x, emb, A, bA, W, U, b, Wout, bout):
    bsz, seqlen = x.shape
    _, hd = emb.shape
    wn = _NUMSTEPS + 1
    assert seqlen >= wn
    xw = x[:, seqlen - wn :].T.reshape(-1).astype(jnp.int32)
    h0 = _sc_gather(emb, xw)
    ac = jnp.concatenate([A[1], A[2], A[3]], axis=1)
    bac = jnp.concatenate([bA[1], bA[2], bA[3]], axis=0).reshape(1, 3 * hd)
    wc = jnp.concatenate([W[0], W[1], W[2]], axis=1)
    uc = jnp.concatenate([U[0], U[1], U[2]], axis=1)
    bc = jnp.concatenate([b[0], b[1], b[2]], axis=0).reshape(1, 3 * hd)
    last = _tc_steps(h0, ac, bac, wc, uc, bc, bsz, _NUMSTEPS)
    return _tc_proj(last, Wout, bout)
